# Initial kernel scaffold; baseline (speedup 1.0000x reference)
import jax
import jax.numpy as jnp
from jax.experimental import pallas as pl


def kernel(x, edge_index, edge_weight, W_lin, att_l, att_r, W_res, pos_emb, Wq, Wk, Wv, W_ff, b_ff):
    raise NotImplementedError("write your pallas kernel here")


# speedup vs baseline: 92.4370x; 92.4370x over previous
import functools

import jax
import jax.numpy as jnp
from jax import lax
from jax.experimental import pallas as pl
from jax.experimental.pallas import tpu as pltpu
from jax.experimental.pallas import tpu_sc as plsc

N = 10000
E = 320000
T = 4
F = 128
H = 8
D = 16

NC = 2
NS = 16
CH = 32
NCHUNKS = E // CH
RPT = 624
DEN_ROWS = 1280
RPTZ = 704

BLK_PRE = 400
BLK_N = 400


def _pre_body(x_ref, wl_ref, wr_ref, al_ref, ar_ref, h_ref, res_ref,
              alt_ref, art_ref):
    x = x_ref[...]
    h = jnp.dot(x, wl_ref[...], preferred_element_type=jnp.float32)
    res = jnp.dot(x, wr_ref[...], preferred_element_type=jnp.float32)
    S = (lax.broadcasted_iota(jnp.int32, (F, H), 0) // D
         == lax.broadcasted_iota(jnp.int32, (F, H), 1)).astype(jnp.float32)
    al = jnp.dot(h * al_ref[...], S, preferred_element_type=jnp.float32)
    ar = jnp.dot(h * ar_ref[...], S, preferred_element_type=jnp.float32)
    h_ref[...] = h
    res_ref[...] = res
    P = (lax.broadcasted_iota(jnp.int32, (H, F), 0)
         == lax.broadcasted_iota(jnp.int32, (H, F), 1)).astype(jnp.float32)
    alt_ref[...] = jnp.dot(al, P, preferred_element_type=jnp.float32)
    art_ref[...] = jnp.dot(ar, P, preferred_element_type=jnp.float32)


def _tc_pre(xf, W_lin, W_res, al_row, ar_row):
    nblk = (T * N) // BLK_PRE
    return pl.pallas_call(
        _pre_body,
        grid=(nblk,),
        in_specs=[
            pl.BlockSpec((BLK_PRE, F), lambda i: (i, 0)),
            pl.BlockSpec((F, F), lambda i: (0, 0)),
            pl.BlockSpec((F, F), lambda i: (0, 0)),
            pl.BlockSpec((1, F), lambda i: (0, 0)),
            pl.BlockSpec((1, F), lambda i: (0, 0)),
        ],
        out_specs=[
            pl.BlockSpec((BLK_PRE, F), lambda i: (i, 0)),
            pl.BlockSpec((BLK_PRE, F), lambda i: (i, 0)),
            pl.BlockSpec((BLK_PRE, F), lambda i: (i, 0)),
            pl.BlockSpec((BLK_PRE, F), lambda i: (i, 0)),
        ],
        out_shape=[
            jax.ShapeDtypeStruct((T * N, F), jnp.float32),
            jax.ShapeDtypeStruct((T * N, F), jnp.float32),
            jax.ShapeDtypeStruct((T * N, F), jnp.float32),
            jax.ShapeDtypeStruct((T * N, F), jnp.float32),
        ],
    )(xf, W_lin, W_res, al_row, ar_row)


def _sc_body(h_hbm, al_hbm, ar_hbm, ei_hbm, ew_hbm, acc_hbm, den_hbm,
             srcb0, dstb0, ewb0, idxc0, alr0, arr0, combo0,
             srcb1, dstb1, ewb1, idxc1, alr1, arr1, combo1,
             zbuf, acc_sh, sem_i, sem_g, sem_s):
    c = lax.axis_index("c")
    s = lax.axis_index("s")
    zf = jnp.zeros((16,), jnp.float32)
    lane = lax.iota(jnp.int32, 16)
    himask = lane < 8

    SRCB = (srcb0, srcb1)
    DSTB = (dstb0, dstb1)
    EWB = (ewb0, ewb1)
    IDXC = (idxc0, idxc1)
    ALR = (alr0, alr1)
    ARR = (arr0, arr1)
    COMBO = (combo0, combo1)

    def _z1(i, _):
        zbuf[i // 8, pl.ds((i % 8) * 16, 16)] = zf
        return _
    lax.fori_loop(0, (16 * F) // 16, _z1, None)

    nchunks = NCHUNKS // NS
    npairs = (nchunks - 1) // 2
    is_last = s == NS - 1

    def _snapshot(tl, _carry):
        t = c * 2 + tl
        tN = t * N
        base_z = s * RPTZ

        def _za(r, _):
            pltpu.async_copy(zbuf, acc_sh.at[pl.ds(base_z + r * 16, 16)],
                             sem_i)
            return _
        lax.fori_loop(0, RPTZ // 16, _za, None)

        @pl.when(is_last)
        def _zero_tail():
            pltpu.sync_copy(zbuf, acc_sh.at[pl.ds(NS * RPTZ, 16)])

        def _zaw(r, _):
            pltpu.make_async_copy(
                zbuf, acc_sh.at[pl.ds(base_z + r * 16, 16)], sem_i).wait()
            return _
        lax.fori_loop(0, RPTZ // 16, _zaw, None)
        plsc.subcore_barrier()

        def issue_l(p, k):
            base = (s + NS * k) * CH
            off = t * (2 * E)
            pltpu.async_copy(ei_hbm.at[pl.ds(off + base, CH)], SRCB[p], sem_i)
            pltpu.async_copy(ei_hbm.at[pl.ds(off + E + base, CH)], DSTB[p],
                             sem_i)
            pltpu.async_copy(ew_hbm.at[pl.ds(t * E + base, CH)], EWB[p],
                             sem_i)

        def wait_l(p):
            for r in (SRCB[p], DSTB[p]):
                pltpu.make_async_copy(ei_hbm.at[pl.ds(0, CH)], r,
                                      sem_i).wait()
            pltpu.make_async_copy(ew_hbm.at[pl.ds(0, CH)], EWB[p],
                                  sem_i).wait()

        def build(p):
            for g in range(CH // 16):
                sl = pl.ds(g * 16, 16)
                d = DSTB[p][sl]
                IDXC[p][sl] = d
                IDXC[p][pl.ds(CH + g * 16, 16)] = (
                    N + lax.shift_right_logical(d, 3))
                SRCB[p][sl] = SRCB[p][sl] + tN
                DSTB[p][sl] = d + tN

        def issue_g(p):
            pltpu.async_copy(al_hbm.at[SRCB[p]], ALR[p], sem_g)
            pltpu.async_copy(ar_hbm.at[DSTB[p]], ARR[p], sem_g)
            pltpu.async_copy(h_hbm.at[SRCB[p]], COMBO[p].at[pl.ds(0, CH)],
                             sem_g)

        def wait_g(p):
            pltpu.make_async_copy(al_hbm.at[SRCB[p]], ALR[p], sem_g).wait()
            pltpu.make_async_copy(ar_hbm.at[DSTB[p]], ARR[p], sem_g).wait()
            pltpu.make_async_copy(h_hbm.at[SRCB[p]],
                                  COMBO[p].at[pl.ds(0, CH)], sem_g).wait()

        def issue_s(p):
            pltpu.async_copy(COMBO[p], acc_sh.at[IDXC[p]], sem_s, add=True)

        def wait_s(p):
            pltpu.make_async_copy(COMBO[p], acc_sh.at[IDXC[p]], sem_s).wait()

        def compute(p):
            def _edge16(g, _):
                ew16 = EWB[p][pl.ds(g * 16, 16)]
                dst16 = IDXC[p][pl.ds(g * 16, 16)]
                for l in range(16):
                    j = g * 16 + l
                    ews = jnp.broadcast_to(ew16[l], (16,))
                    alpha = ews * (ALR[p][j, pl.ds(0, 16)]
                                   + ARR[p][j, pl.ds(0, 16)])
                    ex = jnp.exp(jnp.maximum(alpha, 0.2 * alpha))
                    exm = jnp.where(himask, ex, 0.0)
                    for hh in range(H):
                        spl = jnp.broadcast_to(ex[hh], (16,))
                        sl2 = pl.ds(hh * 16, 16)
                        COMBO[p][j, sl2] = COMBO[p][j, sl2] * spl
                    for q in range(H):
                        COMBO[p][CH + j, pl.ds(q * 16, 16)] = zf
                    COMBO[p][CH + j, pl.ds((dst16[l] % 8) * 16, 16)] = exm
                return _
            lax.fori_loop(0, CH // 16, _edge16, None)

        issue_l(0, 0)
        wait_l(0)
        build(0)
        issue_g(0)
        issue_l(1, 1)

        def _pair(i, _):
            k0 = 2 * i
            wait_g(0)

            @pl.when(i > 0)
            def _ws1():
                wait_s(1)
            wait_l(1)
            build(1)
            issue_g(1)
            compute(0)
            issue_s(0)
            issue_l(0, k0 + 2)
            wait_g(1)
            wait_s(0)
            wait_l(0)
            build(0)
            issue_g(0)
            compute(1)
            issue_s(1)

            @pl.when(i < npairs - 1)
            def _l1():
                issue_l(1, k0 + 3)
            return _
        lax.fori_loop(0, npairs, _pair, None)

        wait_g(0)
        wait_s(1)
        compute(0)
        issue_s(0)
        wait_s(0)

        plsc.subcore_barrier()
        base_n = s * RPT
        pltpu.sync_copy(acc_sh.at[pl.ds(base_n, RPT)],
                        acc_hbm.at[pl.ds(tN + base_n, RPT)])

        @pl.when(is_last)
        def _drain_tail():
            pltpu.sync_copy(acc_sh.at[pl.ds(NS * RPT, 16)],
                            acc_hbm.at[pl.ds(tN + NS * RPT, 16)])

        dpt = DEN_ROWS // NS
        pltpu.sync_copy(acc_sh.at[pl.ds(N + s * dpt, dpt)],
                        den_hbm.at[pl.ds(t * DEN_ROWS + s * dpt, dpt)])
        return _carry
    lax.fori_loop(0, T // NC, _snapshot, None)


def _sc_edge(h_flat, al_tab, ar_tab, ei_flat, ew_flat):
    mesh = plsc.VectorSubcoreMesh(core_axis_name="c", subcore_axis_name="s")
    dbuf = []
    for _ in range(2):
        dbuf += [
            pltpu.VMEM((CH,), jnp.int32),
            pltpu.VMEM((CH,), jnp.int32),
            pltpu.VMEM((CH,), jnp.float32),
            pltpu.VMEM((2 * CH,), jnp.int32),
            pltpu.VMEM((CH, F), jnp.float32),
            pltpu.VMEM((CH, F), jnp.float32),
            pltpu.VMEM((2 * CH, F), jnp.float32),
        ]
    f = functools.partial(
        pl.kernel,
        out_type=(
            jax.ShapeDtypeStruct((T * N, F), jnp.float32),
            jax.ShapeDtypeStruct((T * DEN_ROWS, F), jnp.float32),
        ),
        mesh=mesh,
        scratch_types=dbuf + [
            pltpu.VMEM((16, F), jnp.float32),
            pltpu.VMEM_SHARED((N + DEN_ROWS, F), jnp.float32),
            pltpu.SemaphoreType.DMA,
            pltpu.SemaphoreType.DMA,
            pltpu.SemaphoreType.DMA,
        ],
    )(_sc_body)
    return f(h_flat, al_tab, ar_tab, ei_flat, ew_flat)


def _post_body(acc_ref, den_ref, res_ref, pos_ref, wq_ref, wk_ref, wv_ref,
               wf_ref, bf_ref, out_ref):
    acc = acc_ref[...]
    den = den_ref[...]
    res = res_ref[...]
    B = acc.shape[1]
    B16 = (lax.broadcasted_iota(jnp.int32, (H, F), 1) // D
           == lax.broadcasted_iota(jnp.int32, (H, F), 0)).astype(jnp.float32)
    den128 = jnp.dot(den.reshape(T * B, H), B16,
                     preferred_element_type=jnp.float32).reshape(T, B, F)
    v = acc / (den128 + 1e-16)
    struct = jnp.where(v > 0, v, jnp.exp(v) - 1.0) + res
    ti = struct + pos_ref[...][:, None, :]
    tif = ti.reshape(T * B, F)
    q = jnp.dot(tif, wq_ref[...], preferred_element_type=jnp.float32)
    k = jnp.dot(tif, wk_ref[...], preferred_element_type=jnp.float32)
    vv = jnp.dot(tif, wv_ref[...], preferred_element_type=jnp.float32)
    q = q.reshape(T, B, F)
    k = k.reshape(T, B, F)
    vv = vv.reshape(T, B, F)
    S = (lax.broadcasted_iota(jnp.int32, (F, H), 0) // D
         == lax.broadcasted_iota(jnp.int32, (F, H), 1)).astype(jnp.float32)
    B8 = (lax.broadcasted_iota(jnp.int32, (H, F), 1) // D
          == lax.broadcasted_iota(jnp.int32, (H, F), 0)).astype(jnp.float32)
    scale = 1.0 / (T ** 0.5)
    outs = []
    for t in range(T):
        scs = [jnp.dot(q[t] * k[s], S, preferred_element_type=jnp.float32)
               * scale for s in range(t + 1)]
        m = scs[0]
        for s in range(1, t + 1):
            m = jnp.maximum(m, scs[s])
        exs = [jnp.exp(sc - m) for sc in scs]
        denom = exs[0]
        for s in range(1, t + 1):
            denom = denom + exs[s]
        acc_t = jnp.zeros((B, F), jnp.float32)
        for s in range(t + 1):
            w = jnp.dot(exs[s] / denom, B8, preferred_element_type=jnp.float32)
            acc_t = acc_t + w * vv[s]
        outs.append(acc_t)
    out = jnp.stack(outs, axis=0)
    of = out.reshape(T * B, F)
    ff = jnp.maximum(jnp.dot(of, wf_ref[...], preferred_element_type=jnp.float32)
                     + bf_ref[...], 0.0).reshape(T, B, F) + out
    out_ref[...] = ff + ti


def _tc_post(acc, den, res, pos_emb, Wq, Wk, Wv, W_ff, bf_row):
    nblk = N // BLK_N
    return pl.pallas_call(
        _post_body,
        grid=(nblk,),
        in_specs=[
            pl.BlockSpec((T, BLK_N, F), lambda i: (0, i, 0)),
            pl.BlockSpec((T, BLK_N, H), lambda i: (0, i, 0)),
            pl.BlockSpec((T, BLK_N, F), lambda i: (0, i, 0)),
            pl.BlockSpec((T, F), lambda i: (0, 0)),
            pl.BlockSpec((F, F), lambda i: (0, 0)),
            pl.BlockSpec((F, F), lambda i: (0, 0)),
            pl.BlockSpec((F, F), lambda i: (0, 0)),
            pl.BlockSpec((F, F), lambda i: (0, 0)),
            pl.BlockSpec((1, F), lambda i: (0, 0)),
        ],
        out_specs=pl.BlockSpec((T, BLK_N, F), lambda i: (0, i, 0)),
        out_shape=jax.ShapeDtypeStruct((T, N, F), jnp.float32),
    )(acc, den, res, pos_emb, Wq, Wk, Wv, W_ff, bf_row)


def kernel(x, edge_index, edge_weight, W_lin, att_l, att_r, W_res,
           pos_emb, Wq, Wk, Wv, W_ff, b_ff):
    xf = x.reshape(T * N, F)
    al_row = att_l.reshape(1, F)
    ar_row = att_r.reshape(1, F)
    h_flat, res_flat, al_tab, ar_tab = _tc_pre(xf, W_lin, W_res, al_row, ar_row)
    ei_flat = edge_index.reshape(-1)
    ew_flat = edge_weight.reshape(-1)
    acc, den = _sc_edge(h_flat, al_tab, ar_tab, ei_flat, ew_flat)
    den_unpacked = den.reshape(T, DEN_ROWS, F)[:, :N // 8].reshape(
        T, N, 2 * H)[:, :, :H]
    out = _tc_post(acc.reshape(T, N, F), den_unpacked,
                   res_flat.reshape(T, N, F), pos_emb, Wq, Wk, Wv, W_ff,
                   b_ff.reshape(1, F))
    return jnp.transpose(out, (1, 0, 2))
